# Initial kernel scaffold; baseline (speedup 1.0000x reference)
import jax
import jax.numpy as jnp
from jax.experimental import pallas as pl


def kernel(logits, targets):
    raise NotImplementedError("write your pallas kernel here")


# speedup vs baseline: 92.1289x; 92.1289x over previous
import functools

import jax
import jax.numpy as jnp
from jax.experimental import pallas as pl
from jax.experimental.pallas import tpu as pltpu


def _mrr_body(x_ref, tgt_ref, out_ref, tacc, gt_acc, eq_acc, *, n, w, nb, b_rows):
    p = pl.program_id(0)
    b = pl.program_id(1)
    x = x_ref[...]
    col = jax.lax.broadcasted_iota(jnp.int32, x.shape, 1) + b * w
    tgt = tgt_ref[...]

    @pl.when((p == 0) & (b == 0))
    def _():
        tacc[...] = jnp.zeros_like(tacc)
        gt_acc[...] = jnp.zeros_like(gt_acc)
        eq_acc[...] = jnp.zeros_like(eq_acc)

    @pl.when(p == 0)
    def _():
        tacc[...] += jnp.sum(jnp.where(col == tgt, x, 0.0), axis=1, keepdims=True)

    @pl.when(p == 1)
    def _():
        t = tacc[...]
        valid = col < n
        gt = (x > t) & valid
        eq = (x == t) & (col < tgt)
        gt_acc[...] += jnp.sum(gt.astype(jnp.int32), axis=1, keepdims=True)
        eq_acc[...] += jnp.sum(eq.astype(jnp.int32), axis=1, keepdims=True)

    @pl.when((p == 1) & (b == nb - 1))
    def _():
        rank = (1 + gt_acc[...] + eq_acc[...]).astype(jnp.float32)
        out_ref[0, 0] = jnp.sum(1.0 / rank) * (1.0 / b_rows)


@jax.jit
def kernel(logits, targets):
    if targets.ndim == 2:
        targets = jnp.squeeze(targets, axis=1)
    b_rows, n = logits.shape
    w = 8192
    nb = (n + w - 1) // w
    tgt = targets.astype(jnp.int32).reshape(b_rows, 1)
    out = pl.pallas_call(
        functools.partial(_mrr_body, n=n, w=w, nb=nb, b_rows=b_rows),
        grid=(2, nb),
        in_specs=[
            pl.BlockSpec((b_rows, w), lambda p, b: (0, b)),
            pl.BlockSpec((b_rows, 1), lambda p, b: (0, 0)),
        ],
        out_specs=pl.BlockSpec(memory_space=pltpu.SMEM),
        out_shape=jax.ShapeDtypeStruct((1, 1), jnp.float32),
        scratch_shapes=[
            pltpu.VMEM((b_rows, 1), jnp.float32),
            pltpu.VMEM((b_rows, 1), jnp.int32),
            pltpu.VMEM((b_rows, 1), jnp.int32),
        ],
    )(logits, tgt)
    return out[0, 0]
